# baseline (speedup 1.0000x reference)
import functools

import jax
import jax.numpy as jnp
from jax import lax
from jax.experimental import pallas as pl
from jax.experimental.pallas import tpu as pltpu
from jax.experimental.pallas import tpu_sc as plsc

N = 10000
E = 320000
D = 128
NK = D // 16

NC = 2
NS = 16
NW = NC * NS

NPT = 320
NPAD = NW * NPT

CS = 2000
FL = 2048
FLP = FL + 16
CAP = E + 2 * FL

CB = 256
EPW = E // NW
CG = 400

BR = 2000
NB = E // BR


def _m8(x):
    return pl.multiple_of(x, 8)


def _mesh():
    return plsc.VectorSubcoreMesh(core_axis_name="c", subcore_axis_name="s")


def _wid():
    return lax.axis_index("s") * NC + lax.axis_index("c")


def _sc_g_body(bh_hbm, ch_hbm, src_hbm, dst_hbm, g_hbm, dstb, srcb, bbuf, cbuf,
               sem):
    base = _wid() * EPW

    def chunk(i, carry):
        off = _m8(base + i * CG)
        pltpu.sync_copy(dst_hbm.at[pl.ds(off, CG)], dstb)
        pltpu.sync_copy(src_hbm.at[pl.ds(off, CG)], srcb)
        pltpu.async_copy(bh_hbm.at[dstb], bbuf, sem).wait()
        pltpu.async_copy(ch_hbm.at[srcb], cbuf, sem).wait()

        def row(j, c2):
            for k in range(NK):
                sl = pl.ds(16 * k, 16)
                bbuf[j, sl] = bbuf[j, sl] + cbuf[j, sl]
            return c2

        lax.fori_loop(0, CG, row, 0)
        pltpu.sync_copy(bbuf, g_hbm.at[pl.ds(off, CG)])
        return carry

    lax.fori_loop(0, EPW // CG, chunk, 0)


@jax.jit
def _sc_g(bh, ch, src, dst):
    f = pl.kernel(
        _sc_g_body,
        out_type=jax.ShapeDtypeStruct((E, D), jnp.float32),
        mesh=_mesh(),
        scratch_types=[
            pltpu.VMEM((CG,), jnp.int32),
            pltpu.VMEM((CG,), jnp.int32),
            pltpu.VMEM((CG, D), jnp.float32),
            pltpu.VMEM((CG, D), jnp.float32),
            pltpu.SemaphoreType.DMA,
        ],
    )
    return f(bh, ch, src, dst)


E2 = E // 2
CQ = 640
NCH = E2 // CQ
NWORDS = 8 + N * 8 + 8
SENT = -3.0e38


def _sc_agg_body(e8_hbm, vt_hbm, src_hbm, dst_hbm, aggp_hbm,
                 dstq0, srcq0, eidx0, vidx0, ebuf0, vbuf0,
                 dstq1, srcq1, eidx1, vidx1, ebuf1, vbuf1,
                 aggv, sem_sd, sem_g0, sem_g1):
    c = lax.axis_index("c")
    s = lax.axis_index("s")
    g2 = s // 2
    h = s % 2
    ebase = c * E2
    base_h = 8 - 8 * h
    iota16 = lax.iota(jnp.int32, 16)
    sent16 = jnp.full((16,), SENT, jnp.float32)

    def initstep(i, carry):
        aggv[pl.ds(i * 16, 16)] = sent16
        return carry

    lax.fori_loop(0, NWORDS // 16, initstep, 0)

    def coff(ci):
        return _m8(ebase + jnp.minimum(ci, NCH - 1) * CQ)

    def start_idx(ci, dstq, srcq):
        o = coff(ci)
        pltpu.async_copy(dst_hbm.at[pl.ds(o, CQ)], dstq, sem_sd)
        pltpu.async_copy(src_hbm.at[pl.ds(o, CQ)], srcq, sem_sd)

    def wait_idx(ci, dstq, srcq):
        o = coff(ci)
        pltpu.make_async_copy(dst_hbm.at[pl.ds(o, CQ)], dstq, sem_sd).wait()
        pltpu.make_async_copy(src_hbm.at[pl.ds(o, CQ)], srcq, sem_sd).wait()

    def build_idx(ci, srcq, eidx, vidx):
        o = coff(ci)

        def idxstep(vi, c2):
            sl = pl.ds(vi * 16, 16)
            ev = (o + vi * 16 + iota16) * 8 + g2
            eidx[sl] = ev
            vidx[sl] = srcq[sl] * 8 + g2
            return c2

        lax.fori_loop(0, CQ // 16, idxstep, 0)

    def start_g(eidx, vidx, ebuf, vbuf, sem):
        pltpu.async_copy(e8_hbm.at[eidx], ebuf, sem)
        pltpu.async_copy(vt_hbm.at[vidx], vbuf, sem)

    def wait_g(eidx, vidx, ebuf, vbuf, sem):
        pltpu.make_async_copy(e8_hbm.at[eidx], ebuf, sem).wait()
        pltpu.make_async_copy(vt_hbm.at[vidx], vbuf, sem).wait()

    G = 8

    def make_estep(msk, dstq, ebuf, vbuf):
        def estep(vi, c2):
            dlv = dstq[pl.ds(vi * 16, 16)]
            for q in range(16 // G):
                offs = []
                msgs = []
                for l in range(G):
                    li = q * G + l
                    j = vi * 16 + li
                    x = ebuf[j, pl.ds(0, 16)]
                    v = vbuf[j, pl.ds(0, 16)]
                    msgs.append(jnp.where(msk, x * v, sent16))
                    offs.append(base_h + dlv[li] * 8)
                avals = [aggv[pl.ds(offs[l], 16)] for l in range(G)]
                for l in range(G):
                    aggv[pl.ds(offs[l], 16)] = jnp.maximum(avals[l], msgs[l])
                anyc = offs[0] != offs[0]
                for k in range(1, G):
                    for jj in range(k):
                        anyc = jnp.logical_or(
                            anyc, jnp.abs(offs[k] - offs[jj]) < 16)

                @pl.when(anyc)
                def _replay(offs=offs, msgs=msgs):
                    for l in range(G):
                        aggv[pl.ds(offs[l], 16)] = jnp.maximum(
                            aggv[pl.ds(offs[l], 16)], msgs[l])
            return c2

        return estep

    def process(dstq, ebuf, vbuf):
        @pl.when(h == 0)
        def _even():
            lax.fori_loop(0, CQ // 16, make_estep(iota16 < 8, dstq, ebuf,
                                                  vbuf), 0)

        @pl.when(h == 1)
        def _odd():
            lax.fori_loop(0, CQ // 16, make_estep(iota16 >= 8, dstq, ebuf,
                                                  vbuf), 0)

    wid0 = (dstq0, srcq0, eidx0, vidx0, ebuf0, vbuf0, sem_g0)
    wid1 = (dstq1, srcq1, eidx1, vidx1, ebuf1, vbuf1, sem_g1)
    start_idx(0, dstq0, srcq0)
    wait_idx(0, dstq0, srcq0)
    build_idx(0, srcq0, eidx0, vidx0)
    start_g(eidx0, vidx0, ebuf0, vbuf0, sem_g0)
    start_idx(1, dstq1, srcq1)

    def pair(k, carry):
        c0 = 2 * k
        wait_idx(c0 + 1, dstq1, srcq1)
        build_idx(c0 + 1, srcq1, eidx1, vidx1)
        start_g(eidx1, vidx1, ebuf1, vbuf1, sem_g1)
        wait_g(eidx0, vidx0, ebuf0, vbuf0, sem_g0)
        process(dstq0, ebuf0, vbuf0)
        start_idx(c0 + 2, dstq0, srcq0)
        wait_idx(c0 + 2, dstq0, srcq0)
        build_idx(c0 + 2, srcq0, eidx0, vidx0)
        start_g(eidx0, vidx0, ebuf0, vbuf0, sem_g0)
        wait_g(eidx1, vidx1, ebuf1, vbuf1, sem_g1)
        process(dstq1, ebuf1, vbuf1)
        start_idx(c0 + 3, dstq1, srcq1)
        return carry

    lax.fori_loop(0, NCH // 2, pair, 0)
    wait_idx(NCH + 1, dstq1, srcq1)
    wait_g(eidx0, vidx0, ebuf0, vbuf0, sem_g0)

    t = c * 16 + s
    pltpu.sync_copy(aggv, aggp_hbm.at[pl.ds(_m8(t * NWORDS), NWORDS)])


@jax.jit
def _sc_agg(e8, vt16, src, dst):
    f = pl.kernel(
        _sc_agg_body,
        out_type=jax.ShapeDtypeStruct((NW * NWORDS,), jnp.float32),
        mesh=_mesh(),
        compiler_params=pltpu.CompilerParams(use_tc_tiling_on_sc=False),
        scratch_types=(
            [pltpu.VMEM((CQ,), jnp.int32)] * 4
            + [pltpu.VMEM((CQ, 16), jnp.float32)] * 2
            + [pltpu.VMEM((CQ,), jnp.int32)] * 4
            + [pltpu.VMEM((CQ, 16), jnp.float32)] * 2
            + [pltpu.VMEM((NWORDS,), jnp.float32),
               pltpu.SemaphoreType.DMA,
               pltpu.SemaphoreType.DMA,
               pltpu.SemaphoreType.DMA]
        ),
    )
    return f(e8, vt16, src, dst)


_DN = (((1,), (1,)), ((), ()))


def _tc_mm3_body(h_ref, wv_ref, wb_ref, wc_ref, vh_ref, bh_ref, ch_ref):
    h = h_ref[...]
    vh_ref[...] = lax.dot_general(h, wv_ref[...], _DN,
                                  preferred_element_type=jnp.float32)
    bh_ref[...] = lax.dot_general(h, wb_ref[...], _DN,
                                  preferred_element_type=jnp.float32)
    ch_ref[...] = lax.dot_general(h, wc_ref[...], _DN,
                                  preferred_element_type=jnp.float32)


@jax.jit
def _tc_mm3(h, wv, wb, wc):
    return pl.pallas_call(
        _tc_mm3_body,
        out_shape=[jax.ShapeDtypeStruct((N, D), jnp.float32)] * 3,
    )(h, wv, wb, wc)


def _tc_sig_body(e_ref, w_ref):
    w_ref[...] = jax.nn.sigmoid(e_ref[...])


@jax.jit
def _tc_sig(e):
    return pl.pallas_call(
        _tc_sig_body,
        grid=(NB,),
        in_specs=[pl.BlockSpec((BR, D), lambda i: (i, 0))],
        out_specs=pl.BlockSpec((BR, D), lambda i: (i, 0)),
        out_shape=jax.ShapeDtypeStruct((E, D), jnp.float32),
    )(e)


def _tc_h_body(h_ref, wu_ref, agg0_ref, agg1_ref, out_ref):
    h = h_ref[...]
    a = jnp.maximum(agg0_ref[...], agg1_ref[...])
    agg = jnp.where(a > -1.0e38, a, 0.0)
    y = lax.dot_general(h, wu_ref[...], _DN,
                        preferred_element_type=jnp.float32) + agg
    m = jnp.mean(y, axis=0, keepdims=True)
    v = jnp.mean((y - m) ** 2, axis=0, keepdims=True)
    out_ref[...] = h + jnp.maximum((y - m) / jnp.sqrt(v + 1e-5), 0.0)


@jax.jit
def _tc_h(h, wu, agg0, agg1):
    return pl.pallas_call(
        _tc_h_body,
        out_shape=jax.ShapeDtypeStruct((N, D), jnp.float32),
    )(h, wu, agg0, agg1)


def _tc_e_body(e_ref, g_ref, wa_ref, oe_ref, acc_ref):
    p = pl.program_id(0)
    i = pl.program_id(1)
    y = lax.dot_general(e_ref[...], wa_ref[...], _DN,
                        preferred_element_type=jnp.float32) + g_ref[...]

    @pl.when(p == 0)
    def _accum():
        @pl.when(i == 0)
        def _zero():
            acc_ref[...] = jnp.zeros_like(acc_ref)

        acc_ref[0:1, :] += jnp.sum(y, axis=0, keepdims=True)
        acc_ref[1:2, :] += jnp.sum(y * y, axis=0, keepdims=True)

    @pl.when(p == 1)
    def _apply():
        m = acc_ref[0:1, :] / E
        var = acc_ref[1:2, :] / E - m * m
        r = 1.0 / jnp.sqrt(var + 1e-5)
        oe_ref[...] = e_ref[...] + jnp.maximum((y - m) * r, 0.0)


@jax.jit
def _tc_e(e, g, wa):
    return pl.pallas_call(
        _tc_e_body,
        grid=(2, NB),
        in_specs=[
            pl.BlockSpec((BR, D), lambda p, i: (i, 0)),
            pl.BlockSpec((BR, D), lambda p, i: (i, 0)),
            pl.BlockSpec((D, D), lambda p, i: (0, 0)),
        ],
        out_specs=pl.BlockSpec((BR, D), lambda p, i: (i, 0)),
        out_shape=jax.ShapeDtypeStruct((E, D), jnp.float32),
        scratch_shapes=[pltpu.VMEM((8, D), jnp.float32)],
    )(e, g, wa)


def kernel(h, e, edge_index, W_U, W_V, W_A, W_B, W_C):
    src = edge_index[0]
    dst = edge_index[1]
    vh, bh, ch = _tc_mm3(h, W_V, W_B, W_C)
    g = _sc_g(bh, ch, src, dst)
    w = _tc_sig(e)
    w8 = w.reshape(E * 8, 16)
    vt16 = vh.reshape(N * 8, 16)
    aggp = _sc_agg(w8, vt16, src, dst)
    ar = aggp.reshape(NW, NWORDS)[:, 8:8 + N * 8].reshape(2, 16, N, 8)
    agg0 = ar[0].transpose(1, 0, 2).reshape(N, D)
    agg1 = ar[1].transpose(1, 0, 2).reshape(N, D)
    out_h = _tc_h(h, W_U, agg0, agg1)
    out_e = _tc_e(e, g, W_A)
    return (out_h, out_e)


# speedup vs baseline: 1.1154x; 1.1154x over previous
import functools

import jax
import jax.numpy as jnp
from jax import lax
from jax.experimental import pallas as pl
from jax.experimental.pallas import tpu as pltpu
from jax.experimental.pallas import tpu_sc as plsc

N = 10000
E = 320000
D = 128
NK = D // 16

NC = 2
NS = 16
NW = NC * NS

NPT = 320
NPAD = NW * NPT

CS = 2000
FL = 2048
FLP = FL + 16
CAP = E + 2 * FL

CB = 256
EPW = E // NW
CG = 400

BR = 2000
NB = E // BR


def _m8(x):
    return pl.multiple_of(x, 8)


def _mesh():
    return plsc.VectorSubcoreMesh(core_axis_name="c", subcore_axis_name="s")


def _wid():
    return lax.axis_index("s") * NC + lax.axis_index("c")


def _sc_g_body(bh_hbm, ch_hbm, src_hbm, dst_hbm, g_hbm, dstb, srcb, bbuf, cbuf,
               sem):
    base = _wid() * EPW

    def chunk(i, carry):
        off = _m8(base + i * CG)
        pltpu.sync_copy(dst_hbm.at[pl.ds(off, CG)], dstb)
        pltpu.sync_copy(src_hbm.at[pl.ds(off, CG)], srcb)
        pltpu.async_copy(bh_hbm.at[dstb], bbuf, sem).wait()
        pltpu.async_copy(ch_hbm.at[srcb], cbuf, sem).wait()

        def row(j, c2):
            for k in range(NK):
                sl = pl.ds(16 * k, 16)
                bbuf[j, sl] = bbuf[j, sl] + cbuf[j, sl]
            return c2

        lax.fori_loop(0, CG, row, 0)
        pltpu.sync_copy(bbuf, g_hbm.at[pl.ds(off, CG)])
        return carry

    lax.fori_loop(0, EPW // CG, chunk, 0)


@jax.jit
def _sc_g(bh, ch, src, dst):
    f = pl.kernel(
        _sc_g_body,
        out_type=jax.ShapeDtypeStruct((E, D), jnp.float32),
        mesh=_mesh(),
        scratch_types=[
            pltpu.VMEM((CG,), jnp.int32),
            pltpu.VMEM((CG,), jnp.int32),
            pltpu.VMEM((CG, D), jnp.float32),
            pltpu.VMEM((CG, D), jnp.float32),
            pltpu.SemaphoreType.DMA,
        ],
    )
    return f(bh, ch, src, dst)


E2 = E // 2
CQ = 640
NCH = E2 // CQ
NWORDS = 8 + N * 8 + 8
SENT = -3.0e38


def _sc_agg_body(e8_hbm, vt_hbm, src_hbm, dst_hbm, aggp_hbm,
                 dstq0, srcq0, eidx0, vidx0, ebuf0, vbuf0,
                 dstq1, srcq1, eidx1, vidx1, ebuf1, vbuf1,
                 aggv, sem_sd, sem_g0, sem_g1):
    c = lax.axis_index("c")
    s = lax.axis_index("s")
    g2 = s // 2
    h = s % 2
    ebase = c * E2
    base_h = 8 - 8 * h
    iota16 = lax.iota(jnp.int32, 16)
    sent16 = jnp.full((16,), SENT, jnp.float32)

    def initstep(i, carry):
        aggv[pl.ds(i * 16, 16)] = sent16
        return carry

    lax.fori_loop(0, NWORDS // 16, initstep, 0)

    def coff(ci):
        return _m8(ebase + jnp.minimum(ci, NCH - 1) * CQ)

    def start_idx(ci, dstq, srcq):
        o = coff(ci)
        pltpu.async_copy(dst_hbm.at[pl.ds(o, CQ)], dstq, sem_sd)
        pltpu.async_copy(src_hbm.at[pl.ds(o, CQ)], srcq, sem_sd)

    def wait_idx(ci, dstq, srcq):
        o = coff(ci)
        pltpu.make_async_copy(dst_hbm.at[pl.ds(o, CQ)], dstq, sem_sd).wait()
        pltpu.make_async_copy(src_hbm.at[pl.ds(o, CQ)], srcq, sem_sd).wait()

    def build_idx(ci, srcq, eidx, vidx):
        o = coff(ci)

        def idxstep(vi, c2):
            sl = pl.ds(vi * 16, 16)
            ev = (o + vi * 16 + iota16) * 8 + g2
            eidx[sl] = ev
            vidx[sl] = srcq[sl] * 8 + g2
            return c2

        lax.fori_loop(0, CQ // 16, idxstep, 0)

    def start_g(eidx, vidx, ebuf, vbuf, sem):
        pltpu.async_copy(e8_hbm.at[eidx], ebuf, sem)
        pltpu.async_copy(vt_hbm.at[vidx], vbuf, sem)

    def wait_g(eidx, vidx, ebuf, vbuf, sem):
        pltpu.make_async_copy(e8_hbm.at[eidx], ebuf, sem).wait()
        pltpu.make_async_copy(vt_hbm.at[vidx], vbuf, sem).wait()

    G = 4

    def make_estep(msk, dstq, ebuf, vbuf):
        def estep(vi, c2):
            dlv = dstq[pl.ds(vi * 16, 16)]
            for q in range(16 // G):
                offs = []
                msgs = []
                for l in range(G):
                    li = q * G + l
                    j = vi * 16 + li
                    x = ebuf[j, pl.ds(0, 16)]
                    v = vbuf[j, pl.ds(0, 16)]
                    msgs.append(jnp.where(msk, x * v, sent16))
                    offs.append(base_h + dlv[li] * 8)
                avals = [aggv[pl.ds(offs[l], 16)] for l in range(G)]
                for l in range(G):
                    aggv[pl.ds(offs[l], 16)] = jnp.maximum(avals[l], msgs[l])
                anyc = offs[0] != offs[0]
                for k in range(1, G):
                    for jj in range(k):
                        anyc = jnp.logical_or(
                            anyc, jnp.abs(offs[k] - offs[jj]) < 16)

                @pl.when(anyc)
                def _replay(offs=offs, msgs=msgs):
                    for l in range(G):
                        aggv[pl.ds(offs[l], 16)] = jnp.maximum(
                            aggv[pl.ds(offs[l], 16)], msgs[l])
            return c2

        return estep

    def process(dstq, ebuf, vbuf):
        @pl.when(h == 0)
        def _even():
            lax.fori_loop(0, CQ // 16, make_estep(iota16 < 8, dstq, ebuf,
                                                  vbuf), 0)

        @pl.when(h == 1)
        def _odd():
            lax.fori_loop(0, CQ // 16, make_estep(iota16 >= 8, dstq, ebuf,
                                                  vbuf), 0)

    wid0 = (dstq0, srcq0, eidx0, vidx0, ebuf0, vbuf0, sem_g0)
    wid1 = (dstq1, srcq1, eidx1, vidx1, ebuf1, vbuf1, sem_g1)
    start_idx(0, dstq0, srcq0)
    wait_idx(0, dstq0, srcq0)
    build_idx(0, srcq0, eidx0, vidx0)
    start_g(eidx0, vidx0, ebuf0, vbuf0, sem_g0)
    start_idx(1, dstq1, srcq1)

    def pair(k, carry):
        c0 = 2 * k
        wait_idx(c0 + 1, dstq1, srcq1)
        build_idx(c0 + 1, srcq1, eidx1, vidx1)
        start_g(eidx1, vidx1, ebuf1, vbuf1, sem_g1)
        wait_g(eidx0, vidx0, ebuf0, vbuf0, sem_g0)
        process(dstq0, ebuf0, vbuf0)
        start_idx(c0 + 2, dstq0, srcq0)
        wait_idx(c0 + 2, dstq0, srcq0)
        build_idx(c0 + 2, srcq0, eidx0, vidx0)
        start_g(eidx0, vidx0, ebuf0, vbuf0, sem_g0)
        wait_g(eidx1, vidx1, ebuf1, vbuf1, sem_g1)
        process(dstq1, ebuf1, vbuf1)
        start_idx(c0 + 3, dstq1, srcq1)
        return carry

    lax.fori_loop(0, NCH // 2, pair, 0)
    wait_idx(NCH + 1, dstq1, srcq1)
    wait_g(eidx0, vidx0, ebuf0, vbuf0, sem_g0)

    t = c * 16 + s
    pltpu.sync_copy(aggv, aggp_hbm.at[pl.ds(_m8(t * NWORDS), NWORDS)])


@jax.jit
def _sc_agg(e8, vt16, src, dst):
    f = pl.kernel(
        _sc_agg_body,
        out_type=jax.ShapeDtypeStruct((NW * NWORDS,), jnp.float32),
        mesh=_mesh(),
        compiler_params=pltpu.CompilerParams(use_tc_tiling_on_sc=False),
        scratch_types=(
            [pltpu.VMEM((CQ,), jnp.int32)] * 4
            + [pltpu.VMEM((CQ, 16), jnp.float32)] * 2
            + [pltpu.VMEM((CQ,), jnp.int32)] * 4
            + [pltpu.VMEM((CQ, 16), jnp.float32)] * 2
            + [pltpu.VMEM((NWORDS,), jnp.float32),
               pltpu.SemaphoreType.DMA,
               pltpu.SemaphoreType.DMA,
               pltpu.SemaphoreType.DMA]
        ),
    )
    return f(e8, vt16, src, dst)


_DN = (((1,), (1,)), ((), ()))


def _tc_mm3_body(h_ref, wv_ref, wb_ref, wc_ref, vh_ref, bh_ref, ch_ref):
    h = h_ref[...]
    vh_ref[...] = lax.dot_general(h, wv_ref[...], _DN,
                                  preferred_element_type=jnp.float32)
    bh_ref[...] = lax.dot_general(h, wb_ref[...], _DN,
                                  preferred_element_type=jnp.float32)
    ch_ref[...] = lax.dot_general(h, wc_ref[...], _DN,
                                  preferred_element_type=jnp.float32)


@jax.jit
def _tc_mm3(h, wv, wb, wc):
    return pl.pallas_call(
        _tc_mm3_body,
        out_shape=[jax.ShapeDtypeStruct((N, D), jnp.float32)] * 3,
    )(h, wv, wb, wc)


def _tc_sig_body(e_ref, w_ref):
    w_ref[...] = jax.nn.sigmoid(e_ref[...])


@jax.jit
def _tc_sig(e):
    return pl.pallas_call(
        _tc_sig_body,
        grid=(NB,),
        in_specs=[pl.BlockSpec((BR, D), lambda i: (i, 0))],
        out_specs=pl.BlockSpec((BR, D), lambda i: (i, 0)),
        out_shape=jax.ShapeDtypeStruct((E, D), jnp.float32),
    )(e)


def _tc_h_body(h_ref, wu_ref, agg0_ref, agg1_ref, out_ref):
    h = h_ref[...]
    a = jnp.maximum(agg0_ref[...], agg1_ref[...])
    agg = jnp.where(a > -1.0e38, a, 0.0)
    y = lax.dot_general(h, wu_ref[...], _DN,
                        preferred_element_type=jnp.float32) + agg
    m = jnp.mean(y, axis=0, keepdims=True)
    v = jnp.mean((y - m) ** 2, axis=0, keepdims=True)
    out_ref[...] = h + jnp.maximum((y - m) / jnp.sqrt(v + 1e-5), 0.0)


@jax.jit
def _tc_h(h, wu, agg0, agg1):
    return pl.pallas_call(
        _tc_h_body,
        out_shape=jax.ShapeDtypeStruct((N, D), jnp.float32),
    )(h, wu, agg0, agg1)


def _tc_e_body(e_ref, g_ref, wa_ref, oe_ref, acc_ref):
    p = pl.program_id(0)
    i = pl.program_id(1)
    y = lax.dot_general(e_ref[...], wa_ref[...], _DN,
                        preferred_element_type=jnp.float32) + g_ref[...]

    @pl.when(p == 0)
    def _accum():
        @pl.when(i == 0)
        def _zero():
            acc_ref[...] = jnp.zeros_like(acc_ref)

        acc_ref[0:1, :] += jnp.sum(y, axis=0, keepdims=True)
        acc_ref[1:2, :] += jnp.sum(y * y, axis=0, keepdims=True)

    @pl.when(p == 1)
    def _apply():
        m = acc_ref[0:1, :] / E
        var = acc_ref[1:2, :] / E - m * m
        r = 1.0 / jnp.sqrt(var + 1e-5)
        oe_ref[...] = e_ref[...] + jnp.maximum((y - m) * r, 0.0)


@jax.jit
def _tc_e(e, g, wa):
    return pl.pallas_call(
        _tc_e_body,
        grid=(2, NB),
        in_specs=[
            pl.BlockSpec((BR, D), lambda p, i: (i, 0)),
            pl.BlockSpec((BR, D), lambda p, i: (i, 0)),
            pl.BlockSpec((D, D), lambda p, i: (0, 0)),
        ],
        out_specs=pl.BlockSpec((BR, D), lambda p, i: (i, 0)),
        out_shape=jax.ShapeDtypeStruct((E, D), jnp.float32),
        scratch_shapes=[pltpu.VMEM((8, D), jnp.float32)],
    )(e, g, wa)


def kernel(h, e, edge_index, W_U, W_V, W_A, W_B, W_C):
    src = edge_index[0]
    dst = edge_index[1]
    vh, bh, ch = _tc_mm3(h, W_V, W_B, W_C)
    g = _sc_g(bh, ch, src, dst)
    w = _tc_sig(e)
    w8 = w.reshape(E * 8, 16)
    vt16 = vh.reshape(N * 8, 16)
    aggp = _sc_agg(w8, vt16, src, dst)
    ar = aggp.reshape(NW, NWORDS)[:, 8:8 + N * 8].reshape(2, 16, N, 8)
    agg0 = ar[0].transpose(1, 0, 2).reshape(N, D)
    agg1 = ar[1].transpose(1, 0, 2).reshape(N, D)
    out_h = _tc_h(h, W_U, agg0, agg1)
    out_e = _tc_e(e, g, W_A)
    return (out_h, out_e)


# speedup vs baseline: 1.1637x; 1.0432x over previous
import functools

import jax
import jax.numpy as jnp
from jax import lax
from jax.experimental import pallas as pl
from jax.experimental.pallas import tpu as pltpu
from jax.experimental.pallas import tpu_sc as plsc

N = 10000
E = 320000
D = 128
NK = D // 16

NC = 2
NS = 16
NW = NC * NS

NPT = 320
NPAD = NW * NPT

CS = 2000
FL = 2048
FLP = FL + 16
CAP = E + 2 * FL

CB = 256
EPW = E // NW
CG = 200

BR = 2000
NB = E // BR


def _m8(x):
    return pl.multiple_of(x, 8)


def _mesh():
    return plsc.VectorSubcoreMesh(core_axis_name="c", subcore_axis_name="s")


def _wid():
    return lax.axis_index("s") * NC + lax.axis_index("c")


def _sc_g_body(bh_hbm, ch_hbm, src_hbm, dst_hbm, g_hbm,
               dstb0, srcb0, bbuf0, cbuf0,
               dstb1, srcb1, bbuf1, cbuf1,
               sem_i, sem_g0, sem_g1, sem_w0, sem_w1):
    base = _wid() * EPW
    NCG = EPW // CG

    def off(ci):
        return _m8(base + jnp.minimum(ci, NCG - 1) * CG)

    def start_idx(ci, dstb, srcb):
        pltpu.async_copy(dst_hbm.at[pl.ds(off(ci), CG)], dstb, sem_i)
        pltpu.async_copy(src_hbm.at[pl.ds(off(ci), CG)], srcb, sem_i)

    def wait_idx(ci, dstb, srcb):
        pltpu.make_async_copy(dst_hbm.at[pl.ds(off(ci), CG)], dstb,
                              sem_i).wait()
        pltpu.make_async_copy(src_hbm.at[pl.ds(off(ci), CG)], srcb,
                              sem_i).wait()

    def start_g(dstb, srcb, bbuf, cbuf, sem):
        pltpu.async_copy(bh_hbm.at[dstb], bbuf, sem)
        pltpu.async_copy(ch_hbm.at[srcb], cbuf, sem)

    def wait_g(dstb, srcb, bbuf, cbuf, sem):
        pltpu.make_async_copy(bh_hbm.at[dstb], bbuf, sem).wait()
        pltpu.make_async_copy(ch_hbm.at[srcb], cbuf, sem).wait()

    def add_rows(bbuf, cbuf):
        def row(j, c2):
            for k in range(NK):
                sl = pl.ds(16 * k, 16)
                bbuf[j, sl] = bbuf[j, sl] + cbuf[j, sl]
            return c2

        lax.fori_loop(0, CG, row, 0)

    def start_w(ci, bbuf, sem):
        pltpu.async_copy(bbuf, g_hbm.at[pl.ds(off(ci), CG)], sem)

    def wait_w(ci, bbuf, sem):
        pltpu.make_async_copy(bbuf, g_hbm.at[pl.ds(off(ci), CG)], sem).wait()

    start_idx(0, dstb0, srcb0)
    wait_idx(0, dstb0, srcb0)
    start_g(dstb0, srcb0, bbuf0, cbuf0, sem_g0)
    start_idx(1, dstb1, srcb1)

    def pair2(k, carry):
        c0 = 2 * k

        @pl.when(k > 0)
        def _w1():
            wait_w(c0 - 1, bbuf1, sem_w1)

        wait_idx(c0 + 1, dstb1, srcb1)
        start_g(dstb1, srcb1, bbuf1, cbuf1, sem_g1)
        wait_g(dstb0, srcb0, bbuf0, cbuf0, sem_g0)
        add_rows(bbuf0, cbuf0)
        start_w(c0, bbuf0, sem_w0)
        start_idx(c0 + 2, dstb0, srcb0)

        wait_w(c0, bbuf0, sem_w0)
        wait_idx(c0 + 2, dstb0, srcb0)
        start_g(dstb0, srcb0, bbuf0, cbuf0, sem_g0)
        wait_g(dstb1, srcb1, bbuf1, cbuf1, sem_g1)
        add_rows(bbuf1, cbuf1)
        start_w(c0 + 1, bbuf1, sem_w1)
        start_idx(c0 + 3, dstb1, srcb1)
        return carry

    lax.fori_loop(0, NCG // 2, pair2, 0)
    wait_w(NCG - 1, bbuf1, sem_w1)
    wait_idx(NCG + 1, dstb1, srcb1)
    wait_g(dstb0, srcb0, bbuf0, cbuf0, sem_g0)


@jax.jit
def _sc_g(bh, ch, src, dst):
    f = pl.kernel(
        _sc_g_body,
        out_type=jax.ShapeDtypeStruct((E, D), jnp.float32),
        mesh=_mesh(),
        scratch_types=(
            [pltpu.VMEM((CG,), jnp.int32)] * 2
            + [pltpu.VMEM((CG, D), jnp.float32)] * 2
            + [pltpu.VMEM((CG,), jnp.int32)] * 2
            + [pltpu.VMEM((CG, D), jnp.float32)] * 2
            + [pltpu.SemaphoreType.DMA] * 5
        ),
    )
    return f(bh, ch, src, dst)


E2 = E // 2
CQ = 640
NCH = E2 // CQ
NWORDS = 8 + N * 8 + 8
SENT = -3.0e38


def _sc_agg_body(e8_hbm, vt_hbm, src_hbm, dst_hbm, aggp_hbm,
                 dstq0, srcq0, eidx0, vidx0, ebuf0, vbuf0,
                 dstq1, srcq1, eidx1, vidx1, ebuf1, vbuf1,
                 aggv, sem_sd, sem_g0, sem_g1):
    c = lax.axis_index("c")
    s = lax.axis_index("s")
    g2 = s // 2
    h = s % 2
    ebase = c * E2
    base_h = 8 - 8 * h
    iota16 = lax.iota(jnp.int32, 16)
    sent16 = jnp.full((16,), SENT, jnp.float32)

    def initstep(i, carry):
        aggv[pl.ds(i * 16, 16)] = sent16
        return carry

    lax.fori_loop(0, NWORDS // 16, initstep, 0)

    def coff(ci):
        return _m8(ebase + jnp.minimum(ci, NCH - 1) * CQ)

    def start_idx(ci, dstq, srcq):
        o = coff(ci)
        pltpu.async_copy(dst_hbm.at[pl.ds(o, CQ)], dstq, sem_sd)
        pltpu.async_copy(src_hbm.at[pl.ds(o, CQ)], srcq, sem_sd)

    def wait_idx(ci, dstq, srcq):
        o = coff(ci)
        pltpu.make_async_copy(dst_hbm.at[pl.ds(o, CQ)], dstq, sem_sd).wait()
        pltpu.make_async_copy(src_hbm.at[pl.ds(o, CQ)], srcq, sem_sd).wait()

    def build_idx(ci, srcq, eidx, vidx):
        o = coff(ci)

        def idxstep(vi, c2):
            sl = pl.ds(vi * 16, 16)
            ev = (o + vi * 16 + iota16) * 8 + g2
            eidx[sl] = ev
            vidx[sl] = srcq[sl] * 8 + g2
            return c2

        lax.fori_loop(0, CQ // 16, idxstep, 0)

    def start_g(eidx, vidx, ebuf, vbuf, sem):
        pltpu.async_copy(e8_hbm.at[eidx], ebuf, sem)
        pltpu.async_copy(vt_hbm.at[vidx], vbuf, sem)

    def wait_g(eidx, vidx, ebuf, vbuf, sem):
        pltpu.make_async_copy(e8_hbm.at[eidx], ebuf, sem).wait()
        pltpu.make_async_copy(vt_hbm.at[vidx], vbuf, sem).wait()

    G = 4

    def make_estep(msk, dstq, ebuf, vbuf):
        def estep(vi, c2):
            dlv = dstq[pl.ds(vi * 16, 16)]
            for q in range(16 // G):
                offs = []
                msgs = []
                for l in range(G):
                    li = q * G + l
                    j = vi * 16 + li
                    x = ebuf[j, pl.ds(0, 16)]
                    v = vbuf[j, pl.ds(0, 16)]
                    msgs.append(jnp.where(msk, x * v, sent16))
                    offs.append(base_h + dlv[li] * 8)
                avals = [aggv[pl.ds(offs[l], 16)] for l in range(G)]
                for l in range(G):
                    aggv[pl.ds(offs[l], 16)] = jnp.maximum(avals[l], msgs[l])
                anyc = offs[0] != offs[0]
                for k in range(1, G):
                    for jj in range(k):
                        anyc = jnp.logical_or(
                            anyc, jnp.abs(offs[k] - offs[jj]) < 16)

                @pl.when(anyc)
                def _replay(offs=offs, msgs=msgs):
                    for l in range(G):
                        aggv[pl.ds(offs[l], 16)] = jnp.maximum(
                            aggv[pl.ds(offs[l], 16)], msgs[l])
            return c2

        return estep

    def process(dstq, ebuf, vbuf):
        @pl.when(h == 0)
        def _even():
            lax.fori_loop(0, CQ // 16, make_estep(iota16 < 8, dstq, ebuf,
                                                  vbuf), 0)

        @pl.when(h == 1)
        def _odd():
            lax.fori_loop(0, CQ // 16, make_estep(iota16 >= 8, dstq, ebuf,
                                                  vbuf), 0)

    wid0 = (dstq0, srcq0, eidx0, vidx0, ebuf0, vbuf0, sem_g0)
    wid1 = (dstq1, srcq1, eidx1, vidx1, ebuf1, vbuf1, sem_g1)
    start_idx(0, dstq0, srcq0)
    wait_idx(0, dstq0, srcq0)
    build_idx(0, srcq0, eidx0, vidx0)
    start_g(eidx0, vidx0, ebuf0, vbuf0, sem_g0)
    start_idx(1, dstq1, srcq1)

    def pair(k, carry):
        c0 = 2 * k
        wait_idx(c0 + 1, dstq1, srcq1)
        build_idx(c0 + 1, srcq1, eidx1, vidx1)
        start_g(eidx1, vidx1, ebuf1, vbuf1, sem_g1)
        wait_g(eidx0, vidx0, ebuf0, vbuf0, sem_g0)
        process(dstq0, ebuf0, vbuf0)
        start_idx(c0 + 2, dstq0, srcq0)
        wait_idx(c0 + 2, dstq0, srcq0)
        build_idx(c0 + 2, srcq0, eidx0, vidx0)
        start_g(eidx0, vidx0, ebuf0, vbuf0, sem_g0)
        wait_g(eidx1, vidx1, ebuf1, vbuf1, sem_g1)
        process(dstq1, ebuf1, vbuf1)
        start_idx(c0 + 3, dstq1, srcq1)
        return carry

    lax.fori_loop(0, NCH // 2, pair, 0)
    wait_idx(NCH + 1, dstq1, srcq1)
    wait_g(eidx0, vidx0, ebuf0, vbuf0, sem_g0)

    t = c * 16 + s
    pltpu.sync_copy(aggv, aggp_hbm.at[pl.ds(_m8(t * NWORDS), NWORDS)])


@jax.jit
def _sc_agg(e8, vt16, src, dst):
    f = pl.kernel(
        _sc_agg_body,
        out_type=jax.ShapeDtypeStruct((NW * NWORDS,), jnp.float32),
        mesh=_mesh(),
        compiler_params=pltpu.CompilerParams(use_tc_tiling_on_sc=False),
        scratch_types=(
            [pltpu.VMEM((CQ,), jnp.int32)] * 4
            + [pltpu.VMEM((CQ, 16), jnp.float32)] * 2
            + [pltpu.VMEM((CQ,), jnp.int32)] * 4
            + [pltpu.VMEM((CQ, 16), jnp.float32)] * 2
            + [pltpu.VMEM((NWORDS,), jnp.float32),
               pltpu.SemaphoreType.DMA,
               pltpu.SemaphoreType.DMA,
               pltpu.SemaphoreType.DMA]
        ),
    )
    return f(e8, vt16, src, dst)


_DN = (((1,), (1,)), ((), ()))


def _tc_mm3_body(h_ref, wv_ref, wb_ref, wc_ref, vh_ref, bh_ref, ch_ref):
    h = h_ref[...]
    vh_ref[...] = lax.dot_general(h, wv_ref[...], _DN,
                                  preferred_element_type=jnp.float32)
    bh_ref[...] = lax.dot_general(h, wb_ref[...], _DN,
                                  preferred_element_type=jnp.float32)
    ch_ref[...] = lax.dot_general(h, wc_ref[...], _DN,
                                  preferred_element_type=jnp.float32)


@jax.jit
def _tc_mm3(h, wv, wb, wc):
    return pl.pallas_call(
        _tc_mm3_body,
        out_shape=[jax.ShapeDtypeStruct((N, D), jnp.float32)] * 3,
    )(h, wv, wb, wc)


def _tc_sig_body(e_ref, w_ref):
    w_ref[...] = jax.nn.sigmoid(e_ref[...])


@jax.jit
def _tc_sig(e):
    return pl.pallas_call(
        _tc_sig_body,
        grid=(NB,),
        in_specs=[pl.BlockSpec((BR, D), lambda i: (i, 0))],
        out_specs=pl.BlockSpec((BR, D), lambda i: (i, 0)),
        out_shape=jax.ShapeDtypeStruct((E, D), jnp.float32),
    )(e)


def _tc_h_body(h_ref, wu_ref, agg0_ref, agg1_ref, out_ref):
    h = h_ref[...]
    a = jnp.maximum(agg0_ref[...], agg1_ref[...])
    agg = jnp.where(a > -1.0e38, a, 0.0)
    y = lax.dot_general(h, wu_ref[...], _DN,
                        preferred_element_type=jnp.float32) + agg
    m = jnp.mean(y, axis=0, keepdims=True)
    v = jnp.mean((y - m) ** 2, axis=0, keepdims=True)
    out_ref[...] = h + jnp.maximum((y - m) / jnp.sqrt(v + 1e-5), 0.0)


@jax.jit
def _tc_h(h, wu, agg0, agg1):
    return pl.pallas_call(
        _tc_h_body,
        out_shape=jax.ShapeDtypeStruct((N, D), jnp.float32),
    )(h, wu, agg0, agg1)


def _tc_e_body(e_ref, g_ref, wa_ref, oe_ref, acc_ref):
    p = pl.program_id(0)
    i = pl.program_id(1)
    y = lax.dot_general(e_ref[...], wa_ref[...], _DN,
                        preferred_element_type=jnp.float32) + g_ref[...]

    @pl.when(p == 0)
    def _accum():
        @pl.when(i == 0)
        def _zero():
            acc_ref[...] = jnp.zeros_like(acc_ref)

        acc_ref[0:1, :] += jnp.sum(y, axis=0, keepdims=True)
        acc_ref[1:2, :] += jnp.sum(y * y, axis=0, keepdims=True)

    @pl.when(p == 1)
    def _apply():
        m = acc_ref[0:1, :] / E
        var = acc_ref[1:2, :] / E - m * m
        r = 1.0 / jnp.sqrt(var + 1e-5)
        oe_ref[...] = e_ref[...] + jnp.maximum((y - m) * r, 0.0)


@jax.jit
def _tc_e(e, g, wa):
    return pl.pallas_call(
        _tc_e_body,
        grid=(2, NB),
        in_specs=[
            pl.BlockSpec((BR, D), lambda p, i: (i, 0)),
            pl.BlockSpec((BR, D), lambda p, i: (i, 0)),
            pl.BlockSpec((D, D), lambda p, i: (0, 0)),
        ],
        out_specs=pl.BlockSpec((BR, D), lambda p, i: (i, 0)),
        out_shape=jax.ShapeDtypeStruct((E, D), jnp.float32),
        scratch_shapes=[pltpu.VMEM((8, D), jnp.float32)],
    )(e, g, wa)


def kernel(h, e, edge_index, W_U, W_V, W_A, W_B, W_C):
    src = edge_index[0]
    dst = edge_index[1]
    vh, bh, ch = _tc_mm3(h, W_V, W_B, W_C)
    g = _sc_g(bh, ch, src, dst)
    w = _tc_sig(e)
    w8 = w.reshape(E * 8, 16)
    vt16 = vh.reshape(N * 8, 16)
    aggp = _sc_agg(w8, vt16, src, dst)
    ar = aggp.reshape(NW, NWORDS)[:, 8:8 + N * 8].reshape(2, 16, N, 8)
    agg0 = ar[0].transpose(1, 0, 2).reshape(N, D)
    agg1 = ar[1].transpose(1, 0, 2).reshape(N, D)
    out_h = _tc_h(h, W_U, agg0, agg1)
    out_e = _tc_e(e, g, W_A)
    return (out_h, out_e)


# speedup vs baseline: 1.1641x; 1.0004x over previous
import jax
import jax.numpy as jnp
from jax import lax
from jax.experimental import pallas as pl
from jax.experimental.pallas import tpu as pltpu
from jax.experimental.pallas import tpu_sc as plsc

N = 10000
E = 320000
D = 128
NK = D // 16

NC = 2
NS = 16
NW = NC * NS

NPT = 320
NPAD = NW * NPT

CS = 2000
FL = 2048
FLP = FL + 16
CAP = E + 2 * FL

CB = 256
EPW = E // NW
CG = 200

BR = 2000
NB = E // BR


def _m8(x):
    return pl.multiple_of(x, 8)


def _mesh():
    return plsc.VectorSubcoreMesh(core_axis_name="c", subcore_axis_name="s")


def _wid():
    return lax.axis_index("s") * NC + lax.axis_index("c")


def _sc_g_body(bh_hbm, ch_hbm, src_hbm, dst_hbm, g_hbm,
               dstb0, srcb0, bbuf0, cbuf0,
               dstb1, srcb1, bbuf1, cbuf1,
               sem_i, sem_g0, sem_g1, sem_w0, sem_w1):
    base = _wid() * EPW
    NCG = EPW // CG

    def off(ci):
        return _m8(base + jnp.minimum(ci, NCG - 1) * CG)

    def start_idx(ci, dstb, srcb):
        pltpu.async_copy(dst_hbm.at[pl.ds(off(ci), CG)], dstb, sem_i)
        pltpu.async_copy(src_hbm.at[pl.ds(off(ci), CG)], srcb, sem_i)

    def wait_idx(ci, dstb, srcb):
        pltpu.make_async_copy(dst_hbm.at[pl.ds(off(ci), CG)], dstb,
                              sem_i).wait()
        pltpu.make_async_copy(src_hbm.at[pl.ds(off(ci), CG)], srcb,
                              sem_i).wait()

    def start_g(dstb, srcb, bbuf, cbuf, sem):
        pltpu.async_copy(bh_hbm.at[dstb], bbuf, sem)
        pltpu.async_copy(ch_hbm.at[srcb], cbuf, sem)

    def wait_g(dstb, srcb, bbuf, cbuf, sem):
        pltpu.make_async_copy(bh_hbm.at[dstb], bbuf, sem).wait()
        pltpu.make_async_copy(ch_hbm.at[srcb], cbuf, sem).wait()

    def add_rows(bbuf, cbuf):
        def row(j, c2):
            for k in range(NK):
                sl = pl.ds(16 * k, 16)
                bbuf[j, sl] = bbuf[j, sl] + cbuf[j, sl]
            return c2

        lax.fori_loop(0, CG, row, 0)

    def start_w(ci, bbuf, sem):
        pltpu.async_copy(bbuf, g_hbm.at[pl.ds(off(ci), CG)], sem)

    def wait_w(ci, bbuf, sem):
        pltpu.make_async_copy(bbuf, g_hbm.at[pl.ds(off(ci), CG)], sem).wait()

    start_idx(0, dstb0, srcb0)
    wait_idx(0, dstb0, srcb0)
    start_g(dstb0, srcb0, bbuf0, cbuf0, sem_g0)
    start_idx(1, dstb1, srcb1)

    def pair2(k, carry):
        c0 = 2 * k

        @pl.when(k > 0)
        def _w1():
            wait_w(c0 - 1, bbuf1, sem_w1)

        wait_idx(c0 + 1, dstb1, srcb1)
        start_g(dstb1, srcb1, bbuf1, cbuf1, sem_g1)
        wait_g(dstb0, srcb0, bbuf0, cbuf0, sem_g0)
        add_rows(bbuf0, cbuf0)
        start_w(c0, bbuf0, sem_w0)
        start_idx(c0 + 2, dstb0, srcb0)

        wait_w(c0, bbuf0, sem_w0)
        wait_idx(c0 + 2, dstb0, srcb0)
        start_g(dstb0, srcb0, bbuf0, cbuf0, sem_g0)
        wait_g(dstb1, srcb1, bbuf1, cbuf1, sem_g1)
        add_rows(bbuf1, cbuf1)
        start_w(c0 + 1, bbuf1, sem_w1)
        start_idx(c0 + 3, dstb1, srcb1)
        return carry

    lax.fori_loop(0, NCG // 2, pair2, 0)
    wait_w(NCG - 1, bbuf1, sem_w1)
    wait_idx(NCG + 1, dstb1, srcb1)
    wait_g(dstb0, srcb0, bbuf0, cbuf0, sem_g0)


@jax.jit
def _sc_g(bh, ch, src, dst):
    f = pl.kernel(
        _sc_g_body,
        out_type=jax.ShapeDtypeStruct((E, D), jnp.float32),
        mesh=_mesh(),
        scratch_types=(
            [pltpu.VMEM((CG,), jnp.int32)] * 2
            + [pltpu.VMEM((CG, D), jnp.float32)] * 2
            + [pltpu.VMEM((CG,), jnp.int32)] * 2
            + [pltpu.VMEM((CG, D), jnp.float32)] * 2
            + [pltpu.SemaphoreType.DMA] * 5
        ),
    )
    return f(bh, ch, src, dst)


E2 = E // 2
CQ = 640
NCH = E2 // CQ
NWORDS = 8 + N * 8 + 8
SENT = -3.0e38


def _sc_agg_body(e8_hbm, vt_hbm, src_hbm, dst_hbm, aggp_hbm,
                 dstq0, srcq0, eidx0, vidx0, ebuf0, vbuf0,
                 dstq1, srcq1, eidx1, vidx1, ebuf1, vbuf1,
                 aggv, sem_sd, sem_g0, sem_g1):
    c = lax.axis_index("c")
    s = lax.axis_index("s")
    g2 = s // 2
    h = s % 2
    ebase = c * E2
    base_h = 8 - 8 * h
    iota16 = lax.iota(jnp.int32, 16)
    sent16 = jnp.full((16,), SENT, jnp.float32)

    def initstep(i, carry):
        aggv[pl.ds(i * 16, 16)] = sent16
        return carry

    lax.fori_loop(0, NWORDS // 16, initstep, 0)

    def coff(ci):
        return _m8(ebase + jnp.minimum(ci, NCH - 1) * CQ)

    def start_idx(ci, dstq, srcq):
        o = coff(ci)
        pltpu.async_copy(dst_hbm.at[pl.ds(o, CQ)], dstq, sem_sd)
        pltpu.async_copy(src_hbm.at[pl.ds(o, CQ)], srcq, sem_sd)

    def wait_idx(ci, dstq, srcq):
        o = coff(ci)
        pltpu.make_async_copy(dst_hbm.at[pl.ds(o, CQ)], dstq, sem_sd).wait()
        pltpu.make_async_copy(src_hbm.at[pl.ds(o, CQ)], srcq, sem_sd).wait()

    def build_idx(ci, srcq, eidx, vidx):
        o = coff(ci)

        def idxstep(vi, c2):
            sl = pl.ds(vi * 16, 16)
            ev = (o + vi * 16 + iota16) * 8 + g2
            eidx[sl] = ev
            vidx[sl] = srcq[sl] * 8 + g2
            return c2

        lax.fori_loop(0, CQ // 16, idxstep, 0)

    def start_g(eidx, vidx, ebuf, vbuf, sem):
        pltpu.async_copy(e8_hbm.at[eidx], ebuf, sem)
        pltpu.async_copy(vt_hbm.at[vidx], vbuf, sem)

    def wait_g(eidx, vidx, ebuf, vbuf, sem):
        pltpu.make_async_copy(e8_hbm.at[eidx], ebuf, sem).wait()
        pltpu.make_async_copy(vt_hbm.at[vidx], vbuf, sem).wait()

    G = 4

    def make_estep(msk, dstq, ebuf, vbuf):
        def estep(vi, c2):
            dlv = dstq[pl.ds(vi * 16, 16)]
            for q in range(16 // G):
                offs = []
                msgs = []
                for l in range(G):
                    li = q * G + l
                    j = vi * 16 + li
                    x = ebuf[j, pl.ds(0, 16)]
                    v = vbuf[j, pl.ds(0, 16)]
                    msgs.append(jnp.where(msk, x * v, sent16))
                    offs.append(base_h + dlv[li] * 8)
                avals = [aggv[pl.ds(offs[l], 16)] for l in range(G)]
                for l in range(G):
                    aggv[pl.ds(offs[l], 16)] = jnp.maximum(avals[l], msgs[l])
                anyc = offs[0] != offs[0]
                for k in range(1, G):
                    for jj in range(k):
                        anyc = jnp.logical_or(
                            anyc, jnp.abs(offs[k] - offs[jj]) < 16)

                @pl.when(anyc)
                def _replay(offs=offs, msgs=msgs):
                    for l in range(G):
                        aggv[pl.ds(offs[l], 16)] = jnp.maximum(
                            aggv[pl.ds(offs[l], 16)], msgs[l])
            return c2

        return estep

    def process(dstq, ebuf, vbuf):
        @pl.when(h == 0)
        def _even():
            lax.fori_loop(0, CQ // 16, make_estep(iota16 < 8, dstq, ebuf,
                                                  vbuf), 0)

        @pl.when(h == 1)
        def _odd():
            lax.fori_loop(0, CQ // 16, make_estep(iota16 >= 8, dstq, ebuf,
                                                  vbuf), 0)

    start_idx(0, dstq0, srcq0)
    wait_idx(0, dstq0, srcq0)
    build_idx(0, srcq0, eidx0, vidx0)
    start_g(eidx0, vidx0, ebuf0, vbuf0, sem_g0)
    start_idx(1, dstq1, srcq1)

    def pair(k, carry):
        c0 = 2 * k
        wait_idx(c0 + 1, dstq1, srcq1)
        build_idx(c0 + 1, srcq1, eidx1, vidx1)
        start_g(eidx1, vidx1, ebuf1, vbuf1, sem_g1)
        wait_g(eidx0, vidx0, ebuf0, vbuf0, sem_g0)
        process(dstq0, ebuf0, vbuf0)
        start_idx(c0 + 2, dstq0, srcq0)
        wait_idx(c0 + 2, dstq0, srcq0)
        build_idx(c0 + 2, srcq0, eidx0, vidx0)
        start_g(eidx0, vidx0, ebuf0, vbuf0, sem_g0)
        wait_g(eidx1, vidx1, ebuf1, vbuf1, sem_g1)
        process(dstq1, ebuf1, vbuf1)
        start_idx(c0 + 3, dstq1, srcq1)
        return carry

    lax.fori_loop(0, NCH // 2, pair, 0)
    wait_idx(NCH + 1, dstq1, srcq1)
    wait_g(eidx0, vidx0, ebuf0, vbuf0, sem_g0)

    t = c * 16 + s
    pltpu.sync_copy(aggv, aggp_hbm.at[pl.ds(_m8(t * NWORDS), NWORDS)])


@jax.jit
def _sc_agg(e8, vt16, src, dst):
    f = pl.kernel(
        _sc_agg_body,
        out_type=jax.ShapeDtypeStruct((NW * NWORDS,), jnp.float32),
        mesh=_mesh(),
        compiler_params=pltpu.CompilerParams(use_tc_tiling_on_sc=False),
        scratch_types=(
            [pltpu.VMEM((CQ,), jnp.int32)] * 4
            + [pltpu.VMEM((CQ, 16), jnp.float32)] * 2
            + [pltpu.VMEM((CQ,), jnp.int32)] * 4
            + [pltpu.VMEM((CQ, 16), jnp.float32)] * 2
            + [pltpu.VMEM((NWORDS,), jnp.float32),
               pltpu.SemaphoreType.DMA,
               pltpu.SemaphoreType.DMA,
               pltpu.SemaphoreType.DMA]
        ),
    )
    return f(e8, vt16, src, dst)


_DN = (((1,), (1,)), ((), ()))


def _tc_mm3_body(h_ref, wv_ref, wb_ref, wc_ref, vh_ref, bh_ref, ch_ref):
    h = h_ref[...]
    vh_ref[...] = lax.dot_general(h, wv_ref[...], _DN,
                                  preferred_element_type=jnp.float32)
    bh_ref[...] = lax.dot_general(h, wb_ref[...], _DN,
                                  preferred_element_type=jnp.float32)
    ch_ref[...] = lax.dot_general(h, wc_ref[...], _DN,
                                  preferred_element_type=jnp.float32)


@jax.jit
def _tc_mm3(h, wv, wb, wc):
    return pl.pallas_call(
        _tc_mm3_body,
        out_shape=[jax.ShapeDtypeStruct((N, D), jnp.float32)] * 3,
    )(h, wv, wb, wc)


def _tc_sig_body(e_ref, w_ref):
    w_ref[...] = jax.nn.sigmoid(e_ref[...])


@jax.jit
def _tc_sig(e):
    return pl.pallas_call(
        _tc_sig_body,
        grid=(NB,),
        in_specs=[pl.BlockSpec((BR, D), lambda i: (i, 0))],
        out_specs=pl.BlockSpec((BR, D), lambda i: (i, 0)),
        out_shape=jax.ShapeDtypeStruct((E, D), jnp.float32),
    )(e)


def _tc_h_body(h_ref, wu_ref, agg0_ref, agg1_ref, out_ref):
    h = h_ref[...]
    a = jnp.maximum(agg0_ref[...], agg1_ref[...])
    agg = jnp.where(a > -1.0e38, a, 0.0)
    y = lax.dot_general(h, wu_ref[...], _DN,
                        preferred_element_type=jnp.float32) + agg
    m = jnp.mean(y, axis=0, keepdims=True)
    v = jnp.mean((y - m) ** 2, axis=0, keepdims=True)
    out_ref[...] = h + jnp.maximum((y - m) / jnp.sqrt(v + 1e-5), 0.0)


@jax.jit
def _tc_h(h, wu, agg0, agg1):
    return pl.pallas_call(
        _tc_h_body,
        out_shape=jax.ShapeDtypeStruct((N, D), jnp.float32),
    )(h, wu, agg0, agg1)


def _tc_e_body(e_ref, g_ref, wa_ref, oe_ref, acc_ref):
    p = pl.program_id(0)
    i = pl.program_id(1)
    y = lax.dot_general(e_ref[...], wa_ref[...], _DN,
                        preferred_element_type=jnp.float32) + g_ref[...]

    @pl.when(p == 0)
    def _accum():
        @pl.when(i == 0)
        def _zero():
            acc_ref[...] = jnp.zeros_like(acc_ref)

        acc_ref[0:1, :] += jnp.sum(y, axis=0, keepdims=True)
        acc_ref[1:2, :] += jnp.sum(y * y, axis=0, keepdims=True)

    @pl.when(p == 1)
    def _apply():
        m = acc_ref[0:1, :] / E
        var = acc_ref[1:2, :] / E - m * m
        r = 1.0 / jnp.sqrt(var + 1e-5)
        oe_ref[...] = e_ref[...] + jnp.maximum((y - m) * r, 0.0)


@jax.jit
def _tc_e(e, g, wa):
    return pl.pallas_call(
        _tc_e_body,
        grid=(2, NB),
        in_specs=[
            pl.BlockSpec((BR, D), lambda p, i: (i, 0)),
            pl.BlockSpec((BR, D), lambda p, i: (i, 0)),
            pl.BlockSpec((D, D), lambda p, i: (0, 0)),
        ],
        out_specs=pl.BlockSpec((BR, D), lambda p, i: (i, 0)),
        out_shape=jax.ShapeDtypeStruct((E, D), jnp.float32),
        scratch_shapes=[pltpu.VMEM((8, D), jnp.float32)],
    )(e, g, wa)


def kernel(h, e, edge_index, W_U, W_V, W_A, W_B, W_C):
    src = edge_index[0]
    dst = edge_index[1]
    vh, bh, ch = _tc_mm3(h, W_V, W_B, W_C)
    g = _sc_g(bh, ch, src, dst)
    w = _tc_sig(e)
    w8 = w.reshape(E * 8, 16)
    vt16 = vh.reshape(N * 8, 16)
    aggp = _sc_agg(w8, vt16, src, dst)
    ar = aggp.reshape(NW, NWORDS)[:, 8:8 + N * 8].reshape(2, 16, N, 8)
    agg0 = ar[0].transpose(1, 0, 2).reshape(N, D)
    agg1 = ar[1].transpose(1, 0, 2).reshape(N, D)
    out_h = _tc_h(h, W_U, agg0, agg1)
    out_e = _tc_e(e, g, W_A)
    return (out_h, out_e)
